# Initial kernel scaffold; baseline (speedup 1.0000x reference)
import jax
import jax.numpy as jnp
from jax.experimental import pallas as pl


def kernel(logits, target):
    raise NotImplementedError("write your pallas kernel here")


# speedup vs baseline: 9.9851x; 9.9851x over previous
import functools

import jax
import jax.numpy as jnp
from jax import lax
from jax.experimental import pallas as pl
from jax.experimental.pallas import tpu as pltpu
from jax.experimental.pallas import tpu_sc as plsc

B = 2
C = 4
NPB = 64 * 128 * 128
N = B * NPB
K = max(1, int(0.3 * N))

BLK = 65536


def _loss_body(lg_ref, tg_ref, out_ref):
    x = lg_ref[0]
    t = tg_ref[0]
    m = jnp.max(x, axis=0, keepdims=True)
    s = jnp.sum(jnp.exp(x - m), axis=0, keepdims=True)
    lse = m + jnp.log(s)
    cidx = lax.broadcasted_iota(jnp.int32, (C, BLK), 0)
    sel = jnp.sum(jnp.where(cidx == t, x, 0.0), axis=0, keepdims=True)
    out_ref[0] = jnp.maximum(lse - sel, 0.0)


_loss_call = pl.pallas_call(
    _loss_body,
    grid=(B, NPB // BLK),
    in_specs=[
        pl.BlockSpec((1, C, BLK), lambda b, j: (b, 0, j)),
        pl.BlockSpec((1, 1, BLK), lambda b, j: (b, 0, j)),
    ],
    out_specs=pl.BlockSpec((1, 1, BLK), lambda b, j: (b, 0, j)),
    out_shape=jax.ShapeDtypeStruct((B, 1, NPB), jnp.float32),
)

NC = 2
NS = 16
NW = NC * NS
L = 16
PER_W = N // NW
CH = 16384
NCH = PER_W // CH
NB = 1024
SHIFT1 = 21
SHIFT2 = 11
UNROLL = 8

def _make_hist(use_prefix):
    def body(loss_hbm, *args):
        if use_prefix:
            (pref_hbm, cnt_out, sum_out, stage0, stage1, ctbl, stbl, row,
             pvec, sem0, sem1) = args
        else:
            (cnt_out, sum_out, stage0, stage1, ctbl, stbl, row,
             pvec, sem0, sem1) = args
            pref_hbm = None
        stages = (stage0, stage1)
        wid = lax.axis_index("s") * NC + lax.axis_index("c")
        base = wid * PER_W
        sems = (sem0, sem1)

        zv = jnp.zeros((L,), jnp.float32)

        def zbody(i, carry):
            for u in range(UNROLL):
                off = (i * UNROLL + u) * L
                ctbl[pl.ds(off, L)] = zv
                stbl[pl.ds(off, L)] = zv
            return carry

        lax.fori_loop(0, NB * L // L // UNROLL, zbody, 0)

        if use_prefix:
            pltpu.sync_copy(pref_hbm, pvec)
            pv = pvec[...]

        lane_off = jnp.arange(L, dtype=jnp.int32) * NB
        ones = jnp.full((L,), 1.0, jnp.float32)

        def dma(g):
            return pltpu.make_async_copy(
                loss_hbm.at[pl.ds(base + g * CH, CH)],
                stages[g % 2], sems[g % 2])

        def process(sref):
            def pbody(i, carry):
                for u in range(UNROLL):
                    off = (i * UNROLL + u) * L
                    v = sref[pl.ds(off, L)]
                    bits = lax.bitcast_convert_type(v, jnp.int32)
                    if use_prefix:
                        bucket = jnp.bitwise_and(
                            lax.shift_right_logical(bits, SHIFT2), NB - 1)
                        mask = lax.shift_right_logical(bits, SHIFT1) == pv
                    else:
                        bucket = lax.shift_right_logical(bits, SHIFT1)
                        mask = jnp.full((L,), True)
                    idx = bucket + lane_off
                    plsc.addupdate_scatter(ctbl, [idx], ones, mask=mask)
                    plsc.addupdate_scatter(stbl, [idx], v, mask=mask)
                return carry

            lax.fori_loop(0, CH // L // UNROLL, pbody, 0)

        dma(0).start()
        for g in range(NCH):
            if g + 1 < NCH:
                dma(g + 1).start()
            dma(g).wait()
            process(stages[g % 2])

        for tbl, out in ((ctbl, cnt_out), (stbl, sum_out)):
            def rbody(j, carry, tbl=tbl):
                acc = tbl[pl.ds(j * L, L)]
                for l in range(1, L):
                    acc = acc + tbl[pl.ds(l * NB + j * L, L)]
                row[pl.ds(j * L, L)] = acc
                return carry

            lax.fori_loop(0, NB // L, rbody, 0)
            pltpu.sync_copy(row, out.at[wid])

    return body


@functools.lru_cache(maxsize=1)
def _get_hist_kernels():
    mesh = plsc.VectorSubcoreMesh(core_axis_name="c", subcore_axis_name="s")
    hist_out = [jax.ShapeDtypeStruct((NW, NB), jnp.float32),
                jax.ShapeDtypeStruct((NW, NB), jnp.float32)]
    hist_scratch = [
        pltpu.VMEM((CH,), jnp.float32),
        pltpu.VMEM((CH,), jnp.float32),
        pltpu.VMEM((NB * L,), jnp.float32),
        pltpu.VMEM((NB * L,), jnp.float32),
        pltpu.VMEM((NB,), jnp.float32),
        pltpu.VMEM((L,), jnp.int32),
        pltpu.SemaphoreType.DMA,
        pltpu.SemaphoreType.DMA,
    ]
    cparams = pltpu.CompilerParams(needs_layout_passes=False)
    hist1 = functools.partial(
        pl.kernel, mesh=mesh, out_type=hist_out,
        scratch_types=hist_scratch, compiler_params=cparams)(_make_hist(False))
    hist2 = functools.partial(
        pl.kernel, mesh=mesh, out_type=hist_out,
        scratch_types=hist_scratch, compiler_params=cparams)(_make_hist(True))
    return hist1, hist2


def _scan_core(cnt3, sum3, kneed, sum_above):
    cnt = jnp.sum(cnt3, axis=0)
    sm = jnp.sum(sum3, axis=0)
    tri = (lax.broadcasted_iota(jnp.int32, (128, 128), 0)
           >= lax.broadcasted_iota(jnp.int32, (128, 128), 1)
           ).astype(jnp.float32)
    strict = (lax.broadcasted_iota(jnp.int32, (8, 8), 1)
              > lax.broadcasted_iota(jnp.int32, (8, 8), 0)
              ).astype(jnp.float32)

    def suffix(a):
        srow = jnp.dot(a, tri, preferred_element_type=jnp.float32)
        rt = jnp.sum(a, axis=1).reshape(1, 8)
        r = jnp.sum(strict * rt, axis=1, keepdims=True)
        return srow + r

    s_cnt = suffix(cnt)
    s_sum = suffix(sm)
    fidx = (lax.broadcasted_iota(jnp.int32, (8, 128), 0) * 128
            + lax.broadcasted_iota(jnp.int32, (8, 128), 1)
            ).astype(jnp.float32)
    bsel = jnp.max(jnp.where(s_cnt >= kneed, fidx, -1.0))
    oh = (fidx == bsel).astype(jnp.float32)
    cnt_b = jnp.sum(oh * cnt)
    sum_b = jnp.sum(oh * sm)
    cnt_gt = jnp.sum(oh * s_cnt) - cnt_b
    sum_gt = jnp.sum(oh * s_sum) - sum_b
    k2 = kneed - cnt_gt
    sa2 = sum_above + sum_gt
    return bsel, k2, sa2, cnt_b, sum_b


def _scan1_body(cnt_ref, sum_ref, state_ref, pref_ref):
    bsel, k2, sa2, _, _ = _scan_core(
        cnt_ref[...], sum_ref[...], float(K), 0.0)
    ri = lax.broadcasted_iota(jnp.int32, (8, 128), 0)
    state_ref[...] = jnp.where(ri == 0, k2, sa2)
    pref_ref[...] = jnp.zeros((8, 128), jnp.int32) + bsel.astype(jnp.int32)


def _scan2_body(cnt_ref, sum_ref, st_ref, out_ref):
    kneed = jnp.max(st_ref[0:1, :])
    sum_above = jnp.max(st_ref[1:2, :])
    _, k2, sa2, cnt_b, sum_b = _scan_core(
        cnt_ref[...], sum_ref[...], kneed, sum_above)
    total = sa2 + k2 * (sum_b / cnt_b)
    out_ref[...] = jnp.zeros((8, 128), jnp.float32) + total / float(K)


_scan1 = pl.pallas_call(
    _scan1_body,
    out_shape=[jax.ShapeDtypeStruct((8, 128), jnp.float32),
               jax.ShapeDtypeStruct((8, 128), jnp.int32)],
)

_scan2 = pl.pallas_call(
    _scan2_body,
    out_shape=jax.ShapeDtypeStruct((8, 128), jnp.float32),
)


def kernel(logits, target):
    lg = logits.reshape(B, C, NPB)
    tg = target.astype(jnp.int32).reshape(B, 1, NPB)
    losses = _loss_call(lg, tg).reshape(N)
    _hist1, _hist2 = _get_hist_kernels()
    c1, s1 = _hist1(losses)
    state, pref = _scan1(c1.reshape(NW, 8, 128), s1.reshape(NW, 8, 128))
    pvec = lax.slice(pref, (0, 0), (1, L)).reshape(L)
    c2, s2 = _hist2(losses, pvec)
    res = _scan2(c2.reshape(NW, 8, 128), s2.reshape(NW, 8, 128), state)
    return res[0, 0]
